# initial kernel scaffold (unmeasured)
import jax
import jax.numpy as jnp
from jax import lax
from jax.experimental import pallas as pl
from jax.experimental.pallas import tpu as pltpu

N_DEV = 4


def kernel(x, w_mat):
    m_g, _ = x.shape
    _, n = w_mat.shape
    m_out = m_g // N_DEV

    partial = jnp.dot(x, w_mat, preferred_element_type=jnp.float32)

    def body(part_ref, out_ref, stage, recv, amax_buf,
             send_sems, recv_sems, amax_send_sems, amax_recv_sems,
             local_sem, credit_sem):
        my = lax.axis_index("i")
        left = lax.rem(my + N_DEV - 1, N_DEV)
        right = lax.rem(my + 1, N_DEV)

        barrier = pltpu.get_barrier_semaphore()
        for nbr in (left, right):
            pl.semaphore_signal(barrier, inc=1, device_id=(nbr,),
                                device_id_type=pl.DeviceIdType.MESH)
        pl.semaphore_wait(barrier, 2)

        def load_block(idx):
            cp = pltpu.make_async_copy(
                part_ref.at[pl.ds(idx * m_out, m_out)], stage, local_sem)
            cp.start()
            cp.wait()

        load_block(lax.rem(my + N_DEV - 1, N_DEV))
        for h in range(N_DEV - 1):
            if h >= 1:
                pl.semaphore_wait(credit_sem, 1)
            rdma = pltpu.make_async_remote_copy(
                src_ref=stage, dst_ref=recv,
                send_sem=send_sems.at[h], recv_sem=recv_sems.at[h],
                device_id=(right,), device_id_type=pl.DeviceIdType.MESH)
            rdma.start()
            rdma.wait()
            load_block(lax.rem(my + 2 * N_DEV - h - 2, N_DEV))
            stage[...] = stage[...] + recv[...]
            if h < N_DEV - 2:
                pl.semaphore_signal(credit_sem, inc=1, device_id=(left,),
                                    device_id_type=pl.DeviceIdType.MESH)

        local_amax = jnp.max(jnp.abs(stage[...]))
        amax_buf[pl.ds(my, 1), :] = jnp.full((1, 128), local_amax, jnp.float32)
        sends = []
        for d in range(1, N_DEV):
            tgt = lax.rem(my + d, N_DEV)
            s = pltpu.make_async_remote_copy(
                src_ref=amax_buf.at[pl.ds(my, 1)],
                dst_ref=amax_buf.at[pl.ds(my, 1)],
                send_sem=amax_send_sems.at[d - 1],
                recv_sem=amax_recv_sems.at[my],
                device_id=(tgt,), device_id_type=pl.DeviceIdType.MESH)
            s.start()
            sends.append(s)
        for d in range(1, N_DEV):
            src_pos = lax.rem(my + d, N_DEV)
            r = pltpu.make_async_remote_copy(
                src_ref=amax_buf.at[pl.ds(src_pos, 1)],
                dst_ref=amax_buf.at[pl.ds(src_pos, 1)],
                send_sem=amax_send_sems.at[d - 1],
                recv_sem=amax_recv_sems.at[src_pos],
                device_id=(src_pos,), device_id_type=pl.DeviceIdType.MESH)
            r.wait_recv()
        for s in sends:
            s.wait_send()

        amax_g = jnp.max(amax_buf[...])
        scale = amax_g / 127.0
        q = jnp.clip(jnp.round(stage[...] * (127.0 / amax_g)), -127.0, 127.0)
        out_ref[...] = q * scale

    return pl.pallas_call(
        body,
        out_shape=jax.ShapeDtypeStruct((m_out, n), jnp.float32),
        in_specs=[pl.BlockSpec(memory_space=pltpu.ANY)],
        out_specs=pl.BlockSpec(memory_space=pltpu.VMEM),
        scratch_shapes=[
            pltpu.VMEM((m_out, n), jnp.float32),
            pltpu.VMEM((m_out, n), jnp.float32),
            pltpu.VMEM((N_DEV, 128), jnp.float32),
            pltpu.SemaphoreType.DMA((N_DEV - 1,)),
            pltpu.SemaphoreType.DMA((N_DEV - 1,)),
            pltpu.SemaphoreType.DMA((N_DEV - 1,)),
            pltpu.SemaphoreType.DMA((N_DEV,)),
            pltpu.SemaphoreType.DMA,
            pltpu.SemaphoreType.REGULAR,
        ],
        compiler_params=pltpu.CompilerParams(collective_id=0),
    )(partial)


# baseline (device time: 783359 ns/iter reference)
import jax
import jax.numpy as jnp
from jax import lax
from jax.experimental import pallas as pl
from jax.experimental.pallas import tpu as pltpu

N_DEV = 4
NC = 2


def kernel(x, w_mat):
    m_g, _ = x.shape
    _, n = w_mat.shape
    m_out = m_g // N_DEV
    wc = n // (2 * NC)

    partial = jnp.dot(x, w_mat, preferred_element_type=jnp.float32)

    n_steps = 3 * NC

    def body(part_ref, out_ref, stage_p, recv_p, stage_m, recv_m,
             am_acc, am_send, am_recv,
             send_sems_p, recv_sems_p, send_sems_m, recv_sems_m,
             am_send_sems, am_recv_sems, local_p, local_m,
             credit_p, credit_m, am_credit):
        my = lax.axis_index("i")
        left = lax.rem(my + N_DEV - 1, N_DEV)
        right = lax.rem(my + 1, N_DEV)

        barrier = pltpu.get_barrier_semaphore()
        for nbr in (left, right):
            pl.semaphore_signal(barrier, inc=1, device_id=(nbr,),
                                device_id_type=pl.DeviceIdType.MESH)
        pl.semaphore_wait(barrier, 2)

        def col_of(dirn, c):
            return c * wc if dirn > 0 else (NC + c) * wc

        def load(dirn, stage, sem, blk, c):
            cp = pltpu.make_async_copy(
                part_ref.at[pl.ds(blk * m_out, m_out),
                            pl.ds(col_of(dirn, c), wc)],
                stage, sem)
            cp.start()
            cp.wait()

        am_acc[...] = jnp.zeros((1, 128), jnp.float32)

        load(+1, stage_p, local_p, lax.rem(my + N_DEV - 1, N_DEV), 0)
        load(-1, stage_m, local_m, lax.rem(my + 1, N_DEV), 0)
        for s in range(n_steps):
            c, h = divmod(s, 3)
            if s >= 1:
                pl.semaphore_wait(credit_p, 1)
                pl.semaphore_wait(credit_m, 1)
            r_p = pltpu.make_async_remote_copy(
                src_ref=stage_p, dst_ref=recv_p,
                send_sem=send_sems_p.at[s], recv_sem=recv_sems_p.at[s],
                device_id=(right,), device_id_type=pl.DeviceIdType.MESH)
            r_m = pltpu.make_async_remote_copy(
                src_ref=stage_m, dst_ref=recv_m,
                send_sem=send_sems_m.at[s], recv_sem=recv_sems_m.at[s],
                device_id=(left,), device_id_type=pl.DeviceIdType.MESH)
            r_p.start()
            r_m.start()
            r_p.wait()
            r_m.wait()
            load(+1, stage_p, local_p, lax.rem(my + 2 * N_DEV - h - 2, N_DEV), c)
            stage_p[...] = stage_p[...] + recv_p[...]
            load(-1, stage_m, local_m, lax.rem(my + h + 2, N_DEV), c)
            stage_m[...] = stage_m[...] + recv_m[...]
            if s < n_steps - 1:
                pl.semaphore_signal(credit_p, inc=1, device_id=(left,),
                                    device_id_type=pl.DeviceIdType.MESH)
                pl.semaphore_signal(credit_m, inc=1, device_id=(right,),
                                    device_id_type=pl.DeviceIdType.MESH)
            if h == 2:
                cur = jnp.maximum(jnp.max(jnp.abs(stage_p[...])),
                                  jnp.max(jnp.abs(stage_m[...])))
                am_acc[...] = jnp.maximum(
                    am_acc[...], jnp.full((1, 128), cur, jnp.float32))
                op = pltpu.make_async_copy(
                    stage_p, out_ref.at[:, pl.ds(col_of(+1, c), wc)], local_p)
                om = pltpu.make_async_copy(
                    stage_m, out_ref.at[:, pl.ds(col_of(-1, c), wc)], local_m)
                op.start()
                om.start()
                op.wait()
                om.wait()
                if c < NC - 1:
                    load(+1, stage_p, local_p,
                         lax.rem(my + N_DEV - 1, N_DEV), c + 1)
                    load(-1, stage_m, local_m,
                         lax.rem(my + 1, N_DEV), c + 1)

        am_send[...] = am_acc[...]
        for h in range(N_DEV - 1):
            if h >= 1:
                pl.semaphore_wait(am_credit, 1)
            r_am = pltpu.make_async_remote_copy(
                src_ref=am_send, dst_ref=am_recv,
                send_sem=am_send_sems.at[h], recv_sem=am_recv_sems.at[h],
                device_id=(right,), device_id_type=pl.DeviceIdType.MESH)
            r_am.start()
            r_am.wait()
            am_acc[...] = jnp.maximum(am_acc[...], am_recv[...])
            if h < N_DEV - 2:
                am_send[...] = am_recv[...]
                pl.semaphore_signal(am_credit, inc=1, device_id=(left,),
                                    device_id_type=pl.DeviceIdType.MESH)

        amax_g = jnp.max(am_acc[...])
        scale = amax_g / 127.0
        inv = 127.0 / amax_g

        for t in range(2 * NC):
            cp_in = pltpu.make_async_copy(
                out_ref.at[:, pl.ds(t * wc, wc)], stage_p, local_p)
            cp_in.start()
            cp_in.wait()
            q = jnp.clip(jnp.round(stage_p[...] * inv), -127.0, 127.0)
            stage_p[...] = q * scale
            cp_out = pltpu.make_async_copy(
                stage_p, out_ref.at[:, pl.ds(t * wc, wc)], local_p)
            cp_out.start()
            cp_out.wait()

    return pl.pallas_call(
        body,
        out_shape=jax.ShapeDtypeStruct((m_out, n), jnp.float32),
        in_specs=[pl.BlockSpec(memory_space=pl.ANY)],
        out_specs=pl.BlockSpec(memory_space=pl.ANY),
        scratch_shapes=[
            pltpu.VMEM((m_out, wc), jnp.float32),
            pltpu.VMEM((m_out, wc), jnp.float32),
            pltpu.VMEM((m_out, wc), jnp.float32),
            pltpu.VMEM((m_out, wc), jnp.float32),
            pltpu.VMEM((1, 128), jnp.float32),
            pltpu.VMEM((1, 128), jnp.float32),
            pltpu.VMEM((1, 128), jnp.float32),
            pltpu.SemaphoreType.DMA((n_steps,)),
            pltpu.SemaphoreType.DMA((n_steps,)),
            pltpu.SemaphoreType.DMA((n_steps,)),
            pltpu.SemaphoreType.DMA((n_steps,)),
            pltpu.SemaphoreType.DMA((N_DEV - 1,)),
            pltpu.SemaphoreType.DMA((N_DEV - 1,)),
            pltpu.SemaphoreType.DMA,
            pltpu.SemaphoreType.DMA,
            pltpu.SemaphoreType.REGULAR,
            pltpu.SemaphoreType.REGULAR,
            pltpu.SemaphoreType.REGULAR,
        ],
        compiler_params=pltpu.CompilerParams(
            collective_id=0, vmem_limit_bytes=60 * 1024 * 1024),
    )(partial)


# device time: 701450 ns/iter; 1.1168x vs baseline; 1.1168x over previous
import jax
import jax.numpy as jnp
from jax import lax
from jax.experimental import pallas as pl
from jax.experimental.pallas import tpu as pltpu

N_DEV = 4
NC = 4
N_SENDS = 3 * NC


def kernel(x, w_mat):
    m_g, _ = x.shape
    _, n = w_mat.shape
    m_out = m_g // N_DEV
    wc = n // (2 * NC)

    partial = jnp.dot(x, w_mat, preferred_element_type=jnp.float32)

    def body(part_ref, out_ref, stage_p, recv_p, stage_m, recv_m,
             am_acc, am_send, am_recv,
             send_sems_p, recv_sems_p, send_sems_m, recv_sems_m,
             store_sems_p, store_sems_m, am_send_sems, am_recv_sems,
             local_p, local_m, credit_p, credit_m, am_credit):
        my = lax.axis_index("i")
        left = lax.rem(my + N_DEV - 1, N_DEV)
        right = lax.rem(my + 1, N_DEV)

        barrier = pltpu.get_barrier_semaphore()
        for nbr in (left, right):
            pl.semaphore_signal(barrier, inc=1, device_id=(nbr,),
                                device_id_type=pl.DeviceIdType.MESH)
        pl.semaphore_wait(barrier, 2)

        def col_of(dirn, c):
            return c * wc if dirn > 0 else (NC + c) * wc

        def load(dirn, stage_ref, sem, blk, c):
            cp = pltpu.make_async_copy(
                part_ref.at[pl.ds(blk * m_out, m_out),
                            pl.ds(col_of(dirn, c), wc)],
                stage_ref, sem)
            cp.start()
            cp.wait()

        am_acc[...] = jnp.zeros((1, 128), jnp.float32)

        for c in range(NC):
            load(+1, stage_p.at[c], local_p, lax.rem(my + N_DEV - 1, N_DEV), c)
            load(-1, stage_m.at[c], local_m, lax.rem(my + 1, N_DEV), c)

        descs = [None] * N_SENDS

        def consume(j):
            hj, cj = divmod(j, NC)
            d_p, d_m = descs[j]
            d_p.wait()
            load(+1, stage_p.at[cj], local_p,
                 lax.rem(my + 2 * N_DEV - hj - 2, N_DEV), cj)
            stage_p[cj] = stage_p[cj] + recv_p[j % 2]
            d_m.wait()
            load(-1, stage_m.at[cj], local_m,
                 lax.rem(my + hj + 2, N_DEV), cj)
            stage_m[cj] = stage_m[cj] + recv_m[j % 2]
            if j < N_SENDS - 2:
                pl.semaphore_signal(credit_p, inc=1, device_id=(left,),
                                    device_id_type=pl.DeviceIdType.MESH)
                pl.semaphore_signal(credit_m, inc=1, device_id=(right,),
                                    device_id_type=pl.DeviceIdType.MESH)
            if hj == 2:
                cur = jnp.maximum(jnp.max(jnp.abs(stage_p[cj])),
                                  jnp.max(jnp.abs(stage_m[cj])))
                am_acc[...] = jnp.maximum(
                    am_acc[...], jnp.full((1, 128), cur, jnp.float32))

        for k in range(N_SENDS):
            _, c = divmod(k, NC)
            if k >= 2:
                pl.semaphore_wait(credit_p, 1)
                pl.semaphore_wait(credit_m, 1)
            d_p = pltpu.make_async_remote_copy(
                src_ref=stage_p.at[c], dst_ref=recv_p.at[k % 2],
                send_sem=send_sems_p.at[k], recv_sem=recv_sems_p.at[k],
                device_id=(right,), device_id_type=pl.DeviceIdType.MESH)
            d_m = pltpu.make_async_remote_copy(
                src_ref=stage_m.at[c], dst_ref=recv_m.at[k % 2],
                send_sem=send_sems_m.at[k], recv_sem=recv_sems_m.at[k],
                device_id=(left,), device_id_type=pl.DeviceIdType.MESH)
            d_p.start()
            d_m.start()
            descs[k] = (d_p, d_m)
            if k >= 1:
                consume(k - 1)
        consume(N_SENDS - 1)

        am_send[...] = am_acc[...]
        for h in range(N_DEV - 1):
            if h >= 1:
                pl.semaphore_wait(am_credit, 1)
            r_am = pltpu.make_async_remote_copy(
                src_ref=am_send, dst_ref=am_recv,
                send_sem=am_send_sems.at[h], recv_sem=am_recv_sems.at[h],
                device_id=(right,), device_id_type=pl.DeviceIdType.MESH)
            r_am.start()
            r_am.wait()
            am_acc[...] = jnp.maximum(am_acc[...], am_recv[...])
            if h < N_DEV - 2:
                am_send[...] = am_recv[...]
                pl.semaphore_signal(am_credit, inc=1, device_id=(left,),
                                    device_id_type=pl.DeviceIdType.MESH)

        amax_g = jnp.max(am_acc[...])
        scale = amax_g / 127.0
        inv = 127.0 / amax_g

        stores = []
        for c in range(NC):
            stage_p[c] = jnp.clip(jnp.round(stage_p[c] * inv),
                                  -127.0, 127.0) * scale
            st = pltpu.make_async_copy(
                stage_p.at[c], out_ref.at[:, pl.ds(col_of(+1, c), wc)],
                store_sems_p.at[c])
            st.start()
            stores.append(st)
            stage_m[c] = jnp.clip(jnp.round(stage_m[c] * inv),
                                  -127.0, 127.0) * scale
            st = pltpu.make_async_copy(
                stage_m.at[c], out_ref.at[:, pl.ds(col_of(-1, c), wc)],
                store_sems_m.at[c])
            st.start()
            stores.append(st)
        for st in stores:
            st.wait()

    return pl.pallas_call(
        body,
        out_shape=jax.ShapeDtypeStruct((m_out, n), jnp.float32),
        in_specs=[pl.BlockSpec(memory_space=pl.ANY)],
        out_specs=pl.BlockSpec(memory_space=pl.ANY),
        scratch_shapes=[
            pltpu.VMEM((NC, m_out, wc), jnp.float32),
            pltpu.VMEM((2, m_out, wc), jnp.float32),
            pltpu.VMEM((NC, m_out, wc), jnp.float32),
            pltpu.VMEM((2, m_out, wc), jnp.float32),
            pltpu.VMEM((1, 128), jnp.float32),
            pltpu.VMEM((1, 128), jnp.float32),
            pltpu.VMEM((1, 128), jnp.float32),
            pltpu.SemaphoreType.DMA((N_SENDS,)),
            pltpu.SemaphoreType.DMA((N_SENDS,)),
            pltpu.SemaphoreType.DMA((N_SENDS,)),
            pltpu.SemaphoreType.DMA((N_SENDS,)),
            pltpu.SemaphoreType.DMA((NC,)),
            pltpu.SemaphoreType.DMA((NC,)),
            pltpu.SemaphoreType.DMA((N_DEV - 1,)),
            pltpu.SemaphoreType.DMA((N_DEV - 1,)),
            pltpu.SemaphoreType.DMA,
            pltpu.SemaphoreType.DMA,
            pltpu.SemaphoreType.REGULAR,
            pltpu.SemaphoreType.REGULAR,
            pltpu.SemaphoreType.REGULAR,
        ],
        compiler_params=pltpu.CompilerParams(
            collective_id=0, vmem_limit_bytes=60 * 1024 * 1024),
    )(partial)


# device time: 635324 ns/iter; 1.2330x vs baseline; 1.1041x over previous
import jax
import jax.numpy as jnp
from jax import lax
from jax.experimental import pallas as pl
from jax.experimental.pallas import tpu as pltpu

N_DEV = 4
NC = 8
N_SENDS = 3 * NC
_DEBUG_NO_RING = False
_PROLOGUE_INIT = True


def kernel(x, w_mat):
    m_g, k_sh = x.shape
    _, n = w_mat.shape
    m_out = m_g // N_DEV
    wc = n // (2 * NC)

    def body(x_ref, w_ref, out_ref, stage_p, recv_p, stage_m, recv_m,
             xt_p, wt_p, xt_m, wt_m, am_acc, am_send, am_recv,
             send_sems_p, recv_sems_p, send_sems_m, recv_sems_m,
             store_sems_p, store_sems_m, am_send_sems, am_recv_sems,
             lx_p, lw_p, lx_m, lw_m, credit_p, credit_m, am_credit):
        my = lax.axis_index("i")
        left = lax.rem(my + N_DEV - 1, N_DEV)
        right = lax.rem(my + 1, N_DEV)

        barrier = pltpu.get_barrier_semaphore()
        for nbr in (left, right):
            pl.semaphore_signal(barrier, inc=1, device_id=(nbr,),
                                device_id_type=pl.DeviceIdType.MESH)
        pl.semaphore_wait(barrier, 2)

        def col_of(dirn, c):
            return c * wc if dirn > 0 else (NC + c) * wc

        def start_tiles(dirn, blk, c):
            xt, wt = (xt_p, wt_p) if dirn > 0 else (xt_m, wt_m)
            lx, lw = (lx_p, lw_p) if dirn > 0 else (lx_m, lw_m)
            cx = pltpu.make_async_copy(
                x_ref.at[pl.ds(blk * m_out, m_out), :], xt, lx)
            cw = pltpu.make_async_copy(
                w_ref.at[:, pl.ds(col_of(dirn, c), wc)], wt, lw)
            cx.start()
            cw.start()
            return cx, cw

        def gemm_into(dirn, c, tiles, recv_slot):
            xt, wt = (xt_p, wt_p) if dirn > 0 else (xt_m, wt_m)
            stage = stage_p if dirn > 0 else stage_m
            cx, cw = tiles
            cx.wait()
            cw.wait()
            prod = jnp.dot(xt[...], wt[...],
                           preferred_element_type=jnp.float32)
            if recv_slot is None:
                stage[c] = prod
            else:
                rbuf = recv_p if dirn > 0 else recv_m
                stage[c] = prod + rbuf[recv_slot]

        am_acc[...] = jnp.zeros((1, 128), jnp.float32)

        descs = [None] * N_SENDS

        def consume(j):
            hj, cj = divmod(j, NC)
            d_p, d_m = descs[j]
            tp = start_tiles(+1, lax.rem(my + 2 * N_DEV - hj - 2, N_DEV), cj)
            tm = start_tiles(-1, lax.rem(my + hj + 2, N_DEV), cj)
            d_p.wait()
            gemm_into(+1, cj, tp, j % 2)
            d_m.wait()
            gemm_into(-1, cj, tm, j % 2)
            if j < N_SENDS - 2:
                pl.semaphore_signal(credit_p, inc=1, device_id=(left,),
                                    device_id_type=pl.DeviceIdType.MESH)
                pl.semaphore_signal(credit_m, inc=1, device_id=(right,),
                                    device_id_type=pl.DeviceIdType.MESH)
            if hj == 2:
                cur = jnp.maximum(jnp.max(jnp.abs(stage_p[cj])),
                                  jnp.max(jnp.abs(stage_m[cj])))
                am_acc[...] = jnp.maximum(
                    am_acc[...], jnp.full((1, 128), cur, jnp.float32))

        if _PROLOGUE_INIT:
            for c0 in range(NC):
                tp = start_tiles(+1, lax.rem(my + N_DEV - 1, N_DEV), c0)
                tm = start_tiles(-1, lax.rem(my + 1, N_DEV), c0)
                gemm_into(+1, c0, tp, None)
                gemm_into(-1, c0, tm, None)
        loop_steps = NC if _DEBUG_NO_RING else N_SENDS
        for k in range(loop_steps):
            _, c = divmod(k, NC)
            if not _DEBUG_NO_RING and k >= 2:
                pl.semaphore_wait(credit_p, 1)
                pl.semaphore_wait(credit_m, 1)
            if k < NC and not _PROLOGUE_INIT:
                tp = start_tiles(+1, lax.rem(my + N_DEV - 1, N_DEV), k)
                tm = start_tiles(-1, lax.rem(my + 1, N_DEV), k)
                gemm_into(+1, k, tp, None)
                gemm_into(-1, k, tm, None)
            if _DEBUG_NO_RING:
                continue
            d_p = pltpu.make_async_remote_copy(
                src_ref=stage_p.at[c], dst_ref=recv_p.at[k % 2],
                send_sem=send_sems_p.at[k], recv_sem=recv_sems_p.at[k],
                device_id=(right,), device_id_type=pl.DeviceIdType.MESH)
            d_m = pltpu.make_async_remote_copy(
                src_ref=stage_m.at[c], dst_ref=recv_m.at[k % 2],
                send_sem=send_sems_m.at[k], recv_sem=recv_sems_m.at[k],
                device_id=(left,), device_id_type=pl.DeviceIdType.MESH)
            d_p.start()
            d_m.start()
            descs[k] = (d_p, d_m)
            if k >= 1:
                consume(k - 1)
        if not _DEBUG_NO_RING:
            consume(N_SENDS - 1)
        if _DEBUG_NO_RING:
            cur = jnp.max(jnp.abs(stage_p[0]))
            am_acc[...] = jnp.full((1, 128), cur, jnp.float32)

        am_send[...] = am_acc[...]
        for h in range(0 if _DEBUG_NO_RING else N_DEV - 1):
            if h >= 1:
                pl.semaphore_wait(am_credit, 1)
            r_am = pltpu.make_async_remote_copy(
                src_ref=am_send, dst_ref=am_recv,
                send_sem=am_send_sems.at[h], recv_sem=am_recv_sems.at[h],
                device_id=(right,), device_id_type=pl.DeviceIdType.MESH)
            r_am.start()
            r_am.wait()
            am_acc[...] = jnp.maximum(am_acc[...], am_recv[...])
            if h < N_DEV - 2:
                am_send[...] = am_recv[...]
                pl.semaphore_signal(am_credit, inc=1, device_id=(left,),
                                    device_id_type=pl.DeviceIdType.MESH)

        amax_g = jnp.max(am_acc[...])
        scale = amax_g / 127.0
        inv = 127.0 / amax_g

        stores = []
        for c in range(NC):
            stage_p[c] = jnp.clip(jnp.round(stage_p[c] * inv),
                                  -127.0, 127.0) * scale
            st = pltpu.make_async_copy(
                stage_p.at[c], out_ref.at[:, pl.ds(col_of(+1, c), wc)],
                store_sems_p.at[c])
            st.start()
            stores.append(st)
            stage_m[c] = jnp.clip(jnp.round(stage_m[c] * inv),
                                  -127.0, 127.0) * scale
            st = pltpu.make_async_copy(
                stage_m.at[c], out_ref.at[:, pl.ds(col_of(-1, c), wc)],
                store_sems_m.at[c])
            st.start()
            stores.append(st)
        for st in stores:
            st.wait()

    return pl.pallas_call(
        body,
        out_shape=jax.ShapeDtypeStruct((m_out, n), jnp.float32),
        in_specs=[pl.BlockSpec(memory_space=pl.ANY),
                  pl.BlockSpec(memory_space=pl.ANY)],
        out_specs=pl.BlockSpec(memory_space=pl.ANY),
        scratch_shapes=[
            pltpu.VMEM((NC, m_out, wc), jnp.float32),
            pltpu.VMEM((2, m_out, wc), jnp.float32),
            pltpu.VMEM((NC, m_out, wc), jnp.float32),
            pltpu.VMEM((2, m_out, wc), jnp.float32),
            pltpu.VMEM((m_out, k_sh), jnp.float32),
            pltpu.VMEM((k_sh, wc), jnp.float32),
            pltpu.VMEM((m_out, k_sh), jnp.float32),
            pltpu.VMEM((k_sh, wc), jnp.float32),
            pltpu.VMEM((1, 128), jnp.float32),
            pltpu.VMEM((1, 128), jnp.float32),
            pltpu.VMEM((1, 128), jnp.float32),
            pltpu.SemaphoreType.DMA((N_SENDS,)),
            pltpu.SemaphoreType.DMA((N_SENDS,)),
            pltpu.SemaphoreType.DMA((N_SENDS,)),
            pltpu.SemaphoreType.DMA((N_SENDS,)),
            pltpu.SemaphoreType.DMA((NC,)),
            pltpu.SemaphoreType.DMA((NC,)),
            pltpu.SemaphoreType.DMA((N_DEV - 1,)),
            pltpu.SemaphoreType.DMA((N_DEV - 1,)),
            pltpu.SemaphoreType.DMA,
            pltpu.SemaphoreType.DMA,
            pltpu.SemaphoreType.DMA,
            pltpu.SemaphoreType.DMA,
            pltpu.SemaphoreType.REGULAR,
            pltpu.SemaphoreType.REGULAR,
            pltpu.SemaphoreType.REGULAR,
        ],
        compiler_params=pltpu.CompilerParams(
            collective_id=0, vmem_limit_bytes=62 * 1024 * 1024),
    )(x, w_mat)


# device time: 631134 ns/iter; 1.2412x vs baseline; 1.0066x over previous
import jax
import jax.numpy as jnp
from jax import lax
from jax.experimental import pallas as pl
from jax.experimental.pallas import tpu as pltpu

N_DEV = 4
NC = 8
N_SENDS = 3 * NC
_DEBUG_NO_RING = False
_PROLOGUE_INIT = True


def kernel(x, w_mat):
    m_g, k_sh = x.shape
    _, n = w_mat.shape
    m_out = m_g // N_DEV
    wc = n // (2 * NC)

    def body(x_ref, w_ref, out_ref, stage_p, recv_p, stage_m, recv_m,
             xt_p, wt_p, xt_m, wt_m, am_acc, am_send, am_recv,
             send_sems_p, recv_sems_p, send_sems_m, recv_sems_m,
             store_sems_p, store_sems_m, am_send_sems, am_recv_sems,
             lx_p, lw_p, lx_m, lw_m, credit_p, credit_m):
        my = lax.axis_index("i")
        left = lax.rem(my + N_DEV - 1, N_DEV)
        right = lax.rem(my + 1, N_DEV)

        barrier = pltpu.get_barrier_semaphore()
        for nbr in (left, right):
            pl.semaphore_signal(barrier, inc=1, device_id=(nbr,),
                                device_id_type=pl.DeviceIdType.MESH)
        pl.semaphore_wait(barrier, 2)

        def col_of(dirn, c):
            return c * wc if dirn > 0 else (NC + c) * wc

        def start_tiles(dirn, blk, c):
            xt, wt = (xt_p, wt_p) if dirn > 0 else (xt_m, wt_m)
            lx, lw = (lx_p, lw_p) if dirn > 0 else (lx_m, lw_m)
            cx = pltpu.make_async_copy(
                x_ref.at[pl.ds(blk * m_out, m_out), :], xt, lx)
            cw = pltpu.make_async_copy(
                w_ref.at[:, pl.ds(col_of(dirn, c), wc)], wt, lw)
            cx.start()
            cw.start()
            return cx, cw

        def gemm_into(dirn, c, tiles, recv_slot):
            xt, wt = (xt_p, wt_p) if dirn > 0 else (xt_m, wt_m)
            stage = stage_p if dirn > 0 else stage_m
            cx, cw = tiles
            cx.wait()
            cw.wait()
            prod = jnp.dot(xt[...], wt[...],
                           preferred_element_type=jnp.float32)
            if recv_slot is None:
                stage[c] = prod
            else:
                rbuf = recv_p if dirn > 0 else recv_m
                stage[c] = prod + rbuf[recv_slot]

        am_acc[...] = jnp.zeros((1, 128), jnp.float32)

        descs = [None] * N_SENDS

        def consume(j):
            hj, cj = divmod(j, NC)
            d_p, d_m = descs[j]
            tp = start_tiles(+1, lax.rem(my + 2 * N_DEV - hj - 2, N_DEV), cj)
            tm = start_tiles(-1, lax.rem(my + hj + 2, N_DEV), cj)
            d_p.wait()
            gemm_into(+1, cj, tp, j % 2)
            d_m.wait()
            gemm_into(-1, cj, tm, j % 2)
            if j < N_SENDS - 2:
                pl.semaphore_signal(credit_p, inc=1, device_id=(left,),
                                    device_id_type=pl.DeviceIdType.MESH)
                pl.semaphore_signal(credit_m, inc=1, device_id=(right,),
                                    device_id_type=pl.DeviceIdType.MESH)
            if hj == 2:
                cur = jnp.maximum(jnp.max(jnp.abs(stage_p[cj])),
                                  jnp.max(jnp.abs(stage_m[cj])))
                am_acc[...] = jnp.maximum(
                    am_acc[...], jnp.full((1, 128), cur, jnp.float32))

        if _PROLOGUE_INIT:
            for c0 in range(NC):
                tp = start_tiles(+1, lax.rem(my + N_DEV - 1, N_DEV), c0)
                tm = start_tiles(-1, lax.rem(my + 1, N_DEV), c0)
                gemm_into(+1, c0, tp, None)
                gemm_into(-1, c0, tm, None)
        loop_steps = NC if _DEBUG_NO_RING else N_SENDS
        for k in range(loop_steps):
            _, c = divmod(k, NC)
            if not _DEBUG_NO_RING and k >= 2:
                pl.semaphore_wait(credit_p, 1)
                pl.semaphore_wait(credit_m, 1)
            if k < NC and not _PROLOGUE_INIT:
                tp = start_tiles(+1, lax.rem(my + N_DEV - 1, N_DEV), k)
                tm = start_tiles(-1, lax.rem(my + 1, N_DEV), k)
                gemm_into(+1, k, tp, None)
                gemm_into(-1, k, tm, None)
            if _DEBUG_NO_RING:
                continue
            d_p = pltpu.make_async_remote_copy(
                src_ref=stage_p.at[c], dst_ref=recv_p.at[k % 2],
                send_sem=send_sems_p.at[k], recv_sem=recv_sems_p.at[k],
                device_id=(right,), device_id_type=pl.DeviceIdType.MESH)
            d_m = pltpu.make_async_remote_copy(
                src_ref=stage_m.at[c], dst_ref=recv_m.at[k % 2],
                send_sem=send_sems_m.at[k], recv_sem=recv_sems_m.at[k],
                device_id=(left,), device_id_type=pl.DeviceIdType.MESH)
            d_p.start()
            d_m.start()
            descs[k] = (d_p, d_m)
            if k >= 1:
                consume(k - 1)
        if not _DEBUG_NO_RING:
            consume(N_SENDS - 1)
        if _DEBUG_NO_RING:
            cur = jnp.max(jnp.abs(stage_p[0]))
            am_acc[...] = jnp.full((1, 128), cur, jnp.float32)

        am_send[...] = am_acc[...]
        am_descs = []
        for d in range(1, N_DEV):
            tgt = lax.rem(my + d, N_DEV)
            snd = pltpu.make_async_remote_copy(
                src_ref=am_send, dst_ref=am_recv.at[d - 1],
                send_sem=am_send_sems.at[d - 1],
                recv_sem=am_recv_sems.at[d - 1],
                device_id=(tgt,), device_id_type=pl.DeviceIdType.MESH)
            snd.start()
            am_descs.append(snd)
        for snd in am_descs:
            snd.wait()
        for d in range(1, N_DEV):
            am_acc[...] = jnp.maximum(am_acc[...], am_recv[d - 1])

        amax_g = jnp.max(am_acc[...])
        scale = amax_g / 127.0
        inv = 127.0 / amax_g

        stores = []
        for c in range(NC):
            stage_p[c] = jnp.round(stage_p[c] * inv) * scale
            st = pltpu.make_async_copy(
                stage_p.at[c], out_ref.at[:, pl.ds(col_of(+1, c), wc)],
                store_sems_p.at[c])
            st.start()
            stores.append(st)
            stage_m[c] = jnp.round(stage_m[c] * inv) * scale
            st = pltpu.make_async_copy(
                stage_m.at[c], out_ref.at[:, pl.ds(col_of(-1, c), wc)],
                store_sems_m.at[c])
            st.start()
            stores.append(st)
        for st in stores:
            st.wait()

    return pl.pallas_call(
        body,
        out_shape=jax.ShapeDtypeStruct((m_out, n), jnp.float32),
        in_specs=[pl.BlockSpec(memory_space=pl.ANY),
                  pl.BlockSpec(memory_space=pl.ANY)],
        out_specs=pl.BlockSpec(memory_space=pl.ANY),
        scratch_shapes=[
            pltpu.VMEM((NC, m_out, wc), jnp.float32),
            pltpu.VMEM((2, m_out, wc), jnp.float32),
            pltpu.VMEM((NC, m_out, wc), jnp.float32),
            pltpu.VMEM((2, m_out, wc), jnp.float32),
            pltpu.VMEM((m_out, k_sh), jnp.float32),
            pltpu.VMEM((k_sh, wc), jnp.float32),
            pltpu.VMEM((m_out, k_sh), jnp.float32),
            pltpu.VMEM((k_sh, wc), jnp.float32),
            pltpu.VMEM((1, 128), jnp.float32),
            pltpu.VMEM((1, 128), jnp.float32),
            pltpu.VMEM((N_DEV - 1, 1, 128), jnp.float32),
            pltpu.SemaphoreType.DMA((N_SENDS,)),
            pltpu.SemaphoreType.DMA((N_SENDS,)),
            pltpu.SemaphoreType.DMA((N_SENDS,)),
            pltpu.SemaphoreType.DMA((N_SENDS,)),
            pltpu.SemaphoreType.DMA((NC,)),
            pltpu.SemaphoreType.DMA((NC,)),
            pltpu.SemaphoreType.DMA((N_DEV - 1,)),
            pltpu.SemaphoreType.DMA((N_DEV - 1,)),
            pltpu.SemaphoreType.DMA,
            pltpu.SemaphoreType.DMA,
            pltpu.SemaphoreType.DMA,
            pltpu.SemaphoreType.DMA,
            pltpu.SemaphoreType.REGULAR,
            pltpu.SemaphoreType.REGULAR,
        ],
        compiler_params=pltpu.CompilerParams(
            collective_id=0, vmem_limit_bytes=62 * 1024 * 1024),
    )(x, w_mat)


# device time: 597041 ns/iter; 1.3121x vs baseline; 1.0571x over previous
import jax
import jax.numpy as jnp
from jax import lax
from jax.experimental import pallas as pl
from jax.experimental.pallas import tpu as pltpu

N_DEV = 4
NC = 8
N_SENDS = 3 * NC
_DEBUG_NO_RING = False
_PROLOGUE_INIT = True
_LAZY_TAIL_INIT = True


def kernel(x, w_mat):
    m_g, k_sh = x.shape
    _, n = w_mat.shape
    m_out = m_g // N_DEV
    wc = n // (2 * NC)

    def body(x_ref, w_ref, out_ref, stage_p, recv_p, stage_m, recv_m,
             xt_p, wt_p, xt_m, wt_m, am_acc, am_send, am_recv,
             send_sems_p, recv_sems_p, send_sems_m, recv_sems_m,
             store_sems_p, store_sems_m, am_send_sems, am_recv_sems,
             lx_p, lw_p, lx_m, lw_m, credit_p, credit_m):
        my = lax.axis_index("i")
        left = lax.rem(my + N_DEV - 1, N_DEV)
        right = lax.rem(my + 1, N_DEV)

        barrier = pltpu.get_barrier_semaphore()
        for nbr in (left, right):
            pl.semaphore_signal(barrier, inc=1, device_id=(nbr,),
                                device_id_type=pl.DeviceIdType.MESH)
        pl.semaphore_wait(barrier, 2)

        def col_of(dirn, c):
            return c * wc if dirn > 0 else (NC + c) * wc

        def start_tiles(dirn, blk, c):
            xt, wt = (xt_p, wt_p) if dirn > 0 else (xt_m, wt_m)
            lx, lw = (lx_p, lw_p) if dirn > 0 else (lx_m, lw_m)
            cx = pltpu.make_async_copy(
                x_ref.at[pl.ds(blk * m_out, m_out), :], xt, lx)
            cw = pltpu.make_async_copy(
                w_ref.at[:, pl.ds(col_of(dirn, c), wc)], wt, lw)
            cx.start()
            cw.start()
            return cx, cw

        def gemm_into(dirn, c, tiles, recv_slot):
            xt, wt = (xt_p, wt_p) if dirn > 0 else (xt_m, wt_m)
            stage = stage_p if dirn > 0 else stage_m
            cx, cw = tiles
            cx.wait()
            cw.wait()
            prod = jnp.dot(xt[...], wt[...],
                           preferred_element_type=jnp.float32)
            if recv_slot is None:
                stage[c] = prod
            else:
                rbuf = recv_p if dirn > 0 else recv_m
                stage[c] = prod + rbuf[recv_slot]

        am_acc[...] = jnp.zeros((1, 128), jnp.float32)

        descs = [None] * N_SENDS

        def consume(j):
            hj, cj = divmod(j, NC)
            d_p, d_m = descs[j]
            tp = start_tiles(+1, lax.rem(my + 2 * N_DEV - hj - 2, N_DEV), cj)
            tm = start_tiles(-1, lax.rem(my + hj + 2, N_DEV), cj)
            d_p.wait()
            gemm_into(+1, cj, tp, j % 2)
            d_m.wait()
            gemm_into(-1, cj, tm, j % 2)
            if j < N_SENDS - 2:
                pl.semaphore_signal(credit_p, inc=1, device_id=(left,),
                                    device_id_type=pl.DeviceIdType.MESH)
                pl.semaphore_signal(credit_m, inc=1, device_id=(right,),
                                    device_id_type=pl.DeviceIdType.MESH)
            if hj == 2:
                cur = jnp.maximum(jnp.max(jnp.abs(stage_p[cj])),
                                  jnp.max(jnp.abs(stage_m[cj])))
                am_acc[...] = jnp.maximum(
                    am_acc[...], jnp.full((1, 128), cur, jnp.float32))
            if _LAZY_TAIL_INIT and j + 2 < NC:
                tp = start_tiles(+1, lax.rem(my + N_DEV - 1, N_DEV), j + 2)
                tm = start_tiles(-1, lax.rem(my + 1, N_DEV), j + 2)
                gemm_into(+1, j + 2, tp, None)
                gemm_into(-1, j + 2, tm, None)

        if _PROLOGUE_INIT:
            for c0 in range(2 if _LAZY_TAIL_INIT else NC):
                tp = start_tiles(+1, lax.rem(my + N_DEV - 1, N_DEV), c0)
                tm = start_tiles(-1, lax.rem(my + 1, N_DEV), c0)
                gemm_into(+1, c0, tp, None)
                gemm_into(-1, c0, tm, None)
        loop_steps = NC if _DEBUG_NO_RING else N_SENDS
        for k in range(loop_steps):
            _, c = divmod(k, NC)
            if not _DEBUG_NO_RING and k >= 2:
                pl.semaphore_wait(credit_p, 1)
                pl.semaphore_wait(credit_m, 1)
            if k < NC and not _PROLOGUE_INIT:
                tp = start_tiles(+1, lax.rem(my + N_DEV - 1, N_DEV), k)
                tm = start_tiles(-1, lax.rem(my + 1, N_DEV), k)
                gemm_into(+1, k, tp, None)
                gemm_into(-1, k, tm, None)
            if _DEBUG_NO_RING:
                continue
            d_p = pltpu.make_async_remote_copy(
                src_ref=stage_p.at[c], dst_ref=recv_p.at[k % 2],
                send_sem=send_sems_p.at[k], recv_sem=recv_sems_p.at[k],
                device_id=(right,), device_id_type=pl.DeviceIdType.MESH)
            d_m = pltpu.make_async_remote_copy(
                src_ref=stage_m.at[c], dst_ref=recv_m.at[k % 2],
                send_sem=send_sems_m.at[k], recv_sem=recv_sems_m.at[k],
                device_id=(left,), device_id_type=pl.DeviceIdType.MESH)
            d_p.start()
            d_m.start()
            descs[k] = (d_p, d_m)
            if k >= 1:
                consume(k - 1)
        if not _DEBUG_NO_RING:
            consume(N_SENDS - 1)
        if _DEBUG_NO_RING:
            cur = jnp.max(jnp.abs(stage_p[0]))
            am_acc[...] = jnp.full((1, 128), cur, jnp.float32)

        am_send[...] = am_acc[...]
        am_descs = []
        for d in range(1, N_DEV):
            tgt = lax.rem(my + d, N_DEV)
            snd = pltpu.make_async_remote_copy(
                src_ref=am_send, dst_ref=am_recv.at[d - 1],
                send_sem=am_send_sems.at[d - 1],
                recv_sem=am_recv_sems.at[d - 1],
                device_id=(tgt,), device_id_type=pl.DeviceIdType.MESH)
            snd.start()
            am_descs.append(snd)
        for snd in am_descs:
            snd.wait()
        for d in range(1, N_DEV):
            am_acc[...] = jnp.maximum(am_acc[...], am_recv[d - 1])

        amax_g = jnp.max(am_acc[...])
        scale = amax_g / 127.0
        inv = 127.0 / amax_g

        stores = []
        for c in range(NC):
            stage_p[c] = jnp.round(stage_p[c] * inv) * scale
            st = pltpu.make_async_copy(
                stage_p.at[c], out_ref.at[:, pl.ds(col_of(+1, c), wc)],
                store_sems_p.at[c])
            st.start()
            stores.append(st)
            stage_m[c] = jnp.round(stage_m[c] * inv) * scale
            st = pltpu.make_async_copy(
                stage_m.at[c], out_ref.at[:, pl.ds(col_of(-1, c), wc)],
                store_sems_m.at[c])
            st.start()
            stores.append(st)
        for st in stores:
            st.wait()

    return pl.pallas_call(
        body,
        out_shape=jax.ShapeDtypeStruct((m_out, n), jnp.float32),
        in_specs=[pl.BlockSpec(memory_space=pl.ANY),
                  pl.BlockSpec(memory_space=pl.ANY)],
        out_specs=pl.BlockSpec(memory_space=pl.ANY),
        scratch_shapes=[
            pltpu.VMEM((NC, m_out, wc), jnp.float32),
            pltpu.VMEM((2, m_out, wc), jnp.float32),
            pltpu.VMEM((NC, m_out, wc), jnp.float32),
            pltpu.VMEM((2, m_out, wc), jnp.float32),
            pltpu.VMEM((m_out, k_sh), jnp.float32),
            pltpu.VMEM((k_sh, wc), jnp.float32),
            pltpu.VMEM((m_out, k_sh), jnp.float32),
            pltpu.VMEM((k_sh, wc), jnp.float32),
            pltpu.VMEM((1, 128), jnp.float32),
            pltpu.VMEM((1, 128), jnp.float32),
            pltpu.VMEM((N_DEV - 1, 1, 128), jnp.float32),
            pltpu.SemaphoreType.DMA((N_SENDS,)),
            pltpu.SemaphoreType.DMA((N_SENDS,)),
            pltpu.SemaphoreType.DMA((N_SENDS,)),
            pltpu.SemaphoreType.DMA((N_SENDS,)),
            pltpu.SemaphoreType.DMA((NC,)),
            pltpu.SemaphoreType.DMA((NC,)),
            pltpu.SemaphoreType.DMA((N_DEV - 1,)),
            pltpu.SemaphoreType.DMA((N_DEV - 1,)),
            pltpu.SemaphoreType.DMA,
            pltpu.SemaphoreType.DMA,
            pltpu.SemaphoreType.DMA,
            pltpu.SemaphoreType.DMA,
            pltpu.SemaphoreType.REGULAR,
            pltpu.SemaphoreType.REGULAR,
        ],
        compiler_params=pltpu.CompilerParams(
            collective_id=0, vmem_limit_bytes=62 * 1024 * 1024),
    )(x, w_mat)
